# baseline (speedup 1.0000x reference)
import functools

import jax
import jax.numpy as jnp
from jax.experimental import pallas as pl
from jax.experimental.pallas import tpu as pltpu

_B, _N, _M, _NB = 64, 128, 256, 10
_AF, _BF, _H = 128, 16, 256
_DEPTH = 3


def _wlnet_body(af_ref, bf_ref, ag_ref, bg_ref, mnei_ref, matom_ref,
                w1_ref, wnh_ref, wnb_ref, bn_ref, wah_ref, wan_ref, ba_ref,
                w2a_ref, w2b_ref, w2_ref, out_ref):
    f32 = jnp.float32
    af = af_ref[0]
    bf = bf_ref[0]
    ag = ag_ref[0]
    bg = bg_ref[0]
    mnei = mnei_ref[0]
    matom = matom_ref[0]

    dot = lambda a, b: jax.lax.dot_general(
        a, b, (((1,), (0,)), ((), ())), preferred_element_type=f32)

    oha = (jax.lax.broadcasted_iota(jnp.int32, (_NB * _N, _N), 1) == ag).astype(f32)
    ohb = (jax.lax.broadcasted_iota(jnp.int32, (_NB * _N, _M), 1) == bg).astype(f32)

    def nbsum(x):
        acc = x[0:_N]
        for k in range(1, _NB):
            acc = acc + x[k * _N:(k + 1) * _N]
        return acc

    w1 = w1_ref[...]
    wnh = wnh_ref[...]
    wah = wah_ref[...]
    wan = wan_ref[...]
    bn = bn_ref[...]
    ba = ba_ref[...]

    h = jnp.maximum(dot(af, w1), 0.0)
    bnei = dot(ohb, bf)
    gb = dot(bnei, wnb_ref[...])
    for _ in range(_DEPTH - 1):
        ga = dot(oha, dot(h, wnh))
        r = jnp.maximum(ga + gb + bn, 0.0) * mnei
        nei = nbsum(r)
        h = jnp.maximum(dot(h, wah) + dot(nei, wan) + ba, 0.0)

    a = dot(oha, dot(h, w2a_ref[...]))
    b2 = dot(bnei, w2b_ref[...])
    nei = nbsum(a * b2 * mnei)
    out_ref[0] = dot(h, w2_ref[...]) * nei * matom


@jax.jit
def kernel(atom_feats, bond_feats, atom_graph, bond_graph, num_nbs, n_atoms,
           mask_neis, mask_atoms, W1, Wn, bn, Wa, ba, W2a, W2b, W2):
    del num_nbs, n_atoms
    ag = atom_graph.astype(jnp.int32).transpose(0, 2, 1).reshape(_B, _NB * _N, 1)
    bg = bond_graph.astype(jnp.int32).transpose(0, 2, 1).reshape(_B, _NB * _N, 1)
    mnei = mask_neis.astype(jnp.float32).reshape(_B, _N, _NB).transpose(0, 2, 1)
    mnei = mnei.reshape(_B, _NB * _N, 1)
    matom = mask_atoms.astype(jnp.float32)

    wnh, wnb = Wn[:_H], Wn[_H:]
    wah, wan = Wa[:_H], Wa[_H:]
    bn2 = bn.reshape(1, _H)
    ba2 = ba.reshape(1, _H)

    mol = lambda *blk: pl.BlockSpec(blk, lambda b: (b,) + (0,) * (len(blk) - 1))
    rep = lambda *blk: pl.BlockSpec(blk, lambda b: (0,) * len(blk))

    return pl.pallas_call(
        _wlnet_body,
        grid=(_B,),
        in_specs=[
            mol(1, _N, _AF),
            mol(1, _M, _BF),
            mol(1, _NB * _N, 1),
            mol(1, _NB * _N, 1),
            mol(1, _NB * _N, 1),
            mol(1, _N, 1),
            rep(_AF, _H),
            rep(_H, _H),
            rep(_BF, _H),
            rep(1, _H),
            rep(_H, _H),
            rep(_H, _H),
            rep(1, _H),
            rep(_H, _H),
            rep(_BF, _H),
            rep(_H, _H),
        ],
        out_specs=mol(1, _N, _H),
        out_shape=jax.ShapeDtypeStruct((_B, _N, _H), jnp.float32),
        compiler_params=pltpu.CompilerParams(
            dimension_semantics=("arbitrary",),
        ),
    )(atom_feats, bond_feats, ag, bg, mnei, matom,
      W1, wnh, wnb, bn2, wah, wan, ba2, W2a, W2b, W2)


# speedup vs baseline: 1.0360x; 1.0360x over previous
import jax
import jax.numpy as jnp
from jax.experimental import pallas as pl
from jax.experimental.pallas import tpu as pltpu

_B, _N, _M, _NB = 64, 128, 256, 10
_AF, _BF, _H = 128, 16, 256
_DEPTH = 3
_MOL = 2


def _wlnet_body(af_ref, bf_ref, ag_ref, bg_ref, mnei_ref, matom_ref,
                w1_ref, wnh_ref, wnb_ref, bn_ref, wah_ref, wan_ref, ba_ref,
                w2a_ref, w2b_ref, w2_ref, out_ref):
    f32 = jnp.float32

    dot = lambda a, b: jax.lax.dot_general(
        a, b, (((1,), (0,)), ((), ())), preferred_element_type=f32)

    def nbsum(x):
        acc = x[0:_N]
        for k in range(1, _NB):
            acc = acc + x[k * _N:(k + 1) * _N]
        return acc

    w1 = w1_ref[...]
    wnh = wnh_ref[...]
    wnb = wnb_ref[...]
    wah = wah_ref[...]
    wan = wan_ref[...]
    w2a = w2a_ref[...]
    w2b = w2b_ref[...]
    w2 = w2_ref[...]
    bn = bn_ref[...]
    ba = ba_ref[...]

    for m in range(_MOL):
        af = af_ref[m]
        bf = bf_ref[m]
        ag = ag_ref[m]
        bg = bg_ref[m]
        mnei = mnei_ref[m]
        matom = matom_ref[m]

        oha = (jax.lax.broadcasted_iota(jnp.int32, (_NB * _N, _N), 1)
               == ag).astype(f32)
        ohb = (jax.lax.broadcasted_iota(jnp.int32, (_NB * _N, _M), 1)
               == bg).astype(f32)

        h = jnp.maximum(dot(af, w1), 0.0)
        bnei = dot(ohb, bf) * mnei
        gb = dot(bnei, wnb) + bn + (mnei - 1.0) * 1e30
        for _ in range(_DEPTH - 1):
            ga = dot(oha, dot(h, wnh))
            nei = nbsum(jnp.maximum(ga + gb, 0.0))
            h = jnp.maximum(dot(h, wah) + dot(nei, wan) + ba, 0.0)

        a = dot(oha, dot(h, w2a))
        b2 = dot(bnei, w2b)
        nei = nbsum(a * b2)
        out_ref[m] = dot(h, w2) * nei * matom


@jax.jit
def kernel(atom_feats, bond_feats, atom_graph, bond_graph, num_nbs, n_atoms,
           mask_neis, mask_atoms, W1, Wn, bn, Wa, ba, W2a, W2b, W2):
    del num_nbs, n_atoms
    ag = atom_graph.astype(jnp.int32).transpose(0, 2, 1).reshape(_B, _NB * _N, 1)
    bg = bond_graph.astype(jnp.int32).transpose(0, 2, 1).reshape(_B, _NB * _N, 1)
    mnei = mask_neis.astype(jnp.float32).reshape(_B, _N, _NB).transpose(0, 2, 1)
    mnei = mnei.reshape(_B, _NB * _N, 1)
    matom = mask_atoms.astype(jnp.float32)

    wnh, wnb = Wn[:_H], Wn[_H:]
    wah, wan = Wa[:_H], Wa[_H:]
    bn2 = bn.reshape(1, _H)
    ba2 = ba.reshape(1, _H)

    mol = lambda *blk: pl.BlockSpec((_MOL,) + blk,
                                    lambda b: (b,) + (0,) * len(blk))
    rep = lambda *blk: pl.BlockSpec(blk, lambda b: (0,) * len(blk))

    return pl.pallas_call(
        _wlnet_body,
        grid=(_B // _MOL,),
        in_specs=[
            mol(_N, _AF),
            mol(_M, _BF),
            mol(_NB * _N, 1),
            mol(_NB * _N, 1),
            mol(_NB * _N, 1),
            mol(_N, 1),
            rep(_AF, _H),
            rep(_H, _H),
            rep(_BF, _H),
            rep(1, _H),
            rep(_H, _H),
            rep(_H, _H),
            rep(1, _H),
            rep(_H, _H),
            rep(_BF, _H),
            rep(_H, _H),
        ],
        out_specs=mol(_N, _H),
        out_shape=jax.ShapeDtypeStruct((_B, _N, _H), jnp.float32),
        compiler_params=pltpu.CompilerParams(
            dimension_semantics=("arbitrary",),
        ),
    )(atom_feats, bond_feats, ag, bg, mnei, matom,
      W1, wnh, wnb, bn2, wah, wan, ba2, W2a, W2b, W2)
